# baseline (speedup 1.0000x reference)
import functools

import jax
import jax.numpy as jnp
from jax import lax
from jax.experimental import pallas as pl
from jax.experimental.pallas import tpu as pltpu
from jax.experimental.pallas import tpu_sc as plsc

N_NODES = 10000
N_EDGES = 320000
D = 128

NC = 2
NS = 16
NW = NC * NS
CHUNK = 112
NSTAGE = 4
STCH = 24
NCHUNK = NSTAGE * STCH
EPW = NCHUNK * CHUNK
E_PAD = NW * EPW
NP = 10112
RPT = 640
RPT_LAST = NP - 15 * RPT
DEGP = 10240
ONES_W = CHUNK
ZD_W = 128


def _sc_agg_body(compute_deg, x_hbm, src_hbm, dst_hbm, *refs):
    if compute_deg:
        (agg_hbm, deg_hbm, src_v, dst_v, rows0_v, rows1_v, rows2_v, ones_v,
         zd_v, agg_sh, deg_sh, gsem0, gsem1, gsem2, ssem0, ssem1, ssem2,
         dsem) = refs
    else:
        (agg_hbm, src_v, dst_v, rows0_v, rows1_v, rows2_v, agg_sh,
         gsem0, gsem1, gsem2, ssem0, ssem1, ssem2) = refs
    c = lax.axis_index("c")
    s = lax.axis_index("s")
    wid = c * NS + s
    base = s * RPT

    def _zrow(i, _):
        for j in range(D // 16):
            rows0_v[i, pl.ds(j * 16, 16)] = jnp.zeros((16,), jnp.float32)
        return 0
    lax.fori_loop(0, CHUNK, _zrow, 0)

    def _zero_slice(nrows):
        for k in range(nrows // CHUNK):
            pltpu.sync_copy(rows0_v, agg_sh.at[pl.ds(base + k * CHUNK, CHUNK)])
        rem = nrows % CHUNK
        if rem:
            pltpu.sync_copy(
                rows0_v.at[pl.ds(0, rem)],
                agg_sh.at[pl.ds(base + (nrows // CHUNK) * CHUNK, rem)])

    @pl.when(s < NS - 1)
    def _():
        _zero_slice(RPT)

    @pl.when(s == NS - 1)
    def _():
        _zero_slice(RPT_LAST)

    if compute_deg:
        for j in range(ONES_W // 16):
            ones_v[pl.ds(j * 16, 16)] = jnp.ones((16,), jnp.float32)
        for j in range(ZD_W // 16):
            zd_v[pl.ds(j * 16, 16)] = jnp.zeros((16,), jnp.float32)

        def _zero_deg(nrows):
            for k in range(nrows // ZD_W):
                pltpu.sync_copy(zd_v, deg_sh.at[pl.ds(base + k * ZD_W, ZD_W)])

        @pl.when(s < NS - 1)
        def _():
            _zero_deg(RPT)

        @pl.when(s == NS - 1)
        def _():
            _zero_deg(RPT_LAST)

    plsc.subcore_barrier()

    rows = (rows0_v, rows1_v, rows2_v)
    gsems = (gsem0, gsem1, gsem2)
    ssems = (ssem0, ssem1, ssem2)
    for st in range(NSTAGE):
        pltpu.sync_copy(src_hbm.at[wid, pl.ds(st * STCH, STCH)], src_v)
        pltpu.sync_copy(dst_hbm.at[wid, pl.ds(st * STCH, STCH)], dst_v)
        pltpu.async_copy(x_hbm.at[src_v.at[0]], rows0_v, gsem0)
        pltpu.async_copy(x_hbm.at[src_v.at[1]], rows1_v, gsem1)

        def _triple(t, _):
            for u in range(3):
                j = t * 3 + u
                bu, bp = rows[u], rows[(u + 2) % 3]
                pltpu.make_async_copy(x_hbm.at[src_v.at[j]], bu, gsems[u]).wait()
                pltpu.async_copy(bu, agg_sh.at[dst_v.at[j]], ssems[u], add=True)
                if compute_deg:
                    pltpu.async_copy(ones_v, deg_sh.at[dst_v.at[j]], dsem,
                                     add=True)
                if u == 0:
                    @pl.when(j >= 1)
                    def _():
                        pltpu.make_async_copy(
                            bp, agg_sh.at[dst_v.at[j - 1]],
                            ssems[(u + 2) % 3]).wait()
                else:
                    pltpu.make_async_copy(
                        bp, agg_sh.at[dst_v.at[j - 1]], ssems[(u + 2) % 3]).wait()

                @pl.when(j + 2 < STCH)
                def _():
                    pltpu.async_copy(x_hbm.at[src_v.at[j + 2]], bp, gsems[(u + 2) % 3])
            return 0
        lax.fori_loop(0, STCH // 3, _triple, 0)

        pltpu.make_async_copy(
            rows[(STCH - 1) % 3], agg_sh.at[dst_v.at[STCH - 1]],
            ssems[(STCH - 1) % 3]).wait()
        if compute_deg:
            def _dd(j, _):
                pltpu.make_async_copy(ones_v, deg_sh.at[dst_v.at[j]],
                                      dsem).wait()
                return 0
            lax.fori_loop(0, STCH, _dd, 0)

    plsc.subcore_barrier()

    def _writeback(nrows):
        pltpu.sync_copy(agg_sh.at[pl.ds(base, nrows)],
                        agg_hbm.at[c, pl.ds(base, nrows)])
        if compute_deg:
            pltpu.sync_copy(deg_sh.at[pl.ds(base, nrows)],
                            deg_hbm.at[pl.ds(c * DEGP + base, nrows)])

    @pl.when(s < NS - 1)
    def _():
        _writeback(RPT)

    @pl.when(s == NS - 1)
    def _():
        _writeback(RPT_LAST)


def _make_sc_agg(compute_deg):
    out_type = [jax.ShapeDtypeStruct((NC, NP, D), jnp.float32)]
    if compute_deg:
        out_type.append(jax.ShapeDtypeStruct((NC * DEGP,), jnp.float32))
    scratch = [
        pltpu.VMEM((STCH, CHUNK), jnp.int32),
        pltpu.VMEM((STCH, CHUNK), jnp.int32),
        pltpu.VMEM((CHUNK, D), jnp.float32),
        pltpu.VMEM((CHUNK, D), jnp.float32),
        pltpu.VMEM((CHUNK, D), jnp.float32),
    ]
    if compute_deg:
        scratch += [
            pltpu.VMEM((ONES_W,), jnp.float32),
            pltpu.VMEM((ZD_W,), jnp.float32),
        ]
    scratch += [pltpu.VMEM_SHARED((NP, D), jnp.float32)]
    if compute_deg:
        scratch += [pltpu.VMEM_SHARED((NP,), jnp.float32)]
    scratch += [pltpu.SemaphoreType.DMA] * (7 if compute_deg else 6)
    return pl.kernel(
        functools.partial(_sc_agg_body, compute_deg),
        out_type=out_type,
        mesh=plsc.VectorSubcoreMesh(
            core_axis_name="c", subcore_axis_name="s",
            num_cores=NC, num_subcores=NS,
        ),
        scratch_types=scratch,
    )


_TCB = 1024


def _tc_dense_body(relu, agg_ref, deg_ref, x_ref, wl_ref, bl_ref, wr_ref, o_ref):
    i = pl.program_id(0)
    d2 = deg_ref[:, pl.ds(i * _TCB, _TCB)]
    invd = 1.0 / jnp.maximum(d2[0] + d2[1], 1.0)
    mean = (agg_ref[0] + agg_ref[1]) * invd[:, None]
    acc = jnp.dot(mean, wl_ref[...], preferred_element_type=jnp.float32)
    acc += jnp.dot(x_ref[...], wr_ref[...], preferred_element_type=jnp.float32)
    acc += bl_ref[...]
    if relu:
        acc = jnp.maximum(acc, 0.0)
    o_ref[...] = acc


def _tc_dense(relu, agg, deg, x, wlT, bl, wrT):
    grid = -(-N_NODES // _TCB)
    return pl.pallas_call(
        functools.partial(_tc_dense_body, relu),
        grid=(grid,),
        in_specs=[
            pl.BlockSpec((NC, _TCB, D), lambda i: (0, i, 0)),
            pl.BlockSpec((NC, DEGP), lambda i: (0, 0)),
            pl.BlockSpec((_TCB, D), lambda i: (i, 0)),
            pl.BlockSpec((D, D), lambda i: (0, 0)),
            pl.BlockSpec((1, D), lambda i: (0, 0)),
            pl.BlockSpec((D, D), lambda i: (0, 0)),
        ],
        out_specs=pl.BlockSpec((_TCB, D), lambda i: (i, 0)),
        out_shape=jax.ShapeDtypeStruct((N_NODES, D), jnp.float32),
    )(agg, deg, x, wlT, bl, wrT)


def kernel(x, edge_index, W1l, b1l, W1r, W2l, b2l, W2r):
    ei = edge_index.astype(jnp.int32)
    npad = E_PAD - N_EDGES
    pad_src = jnp.arange(npad, dtype=jnp.int32) % N_NODES
    src = jnp.concatenate([ei[0], pad_src]).reshape(NW, NCHUNK, CHUNK)
    pad_dst = N_NODES + jnp.arange(npad, dtype=jnp.int32) % (NP - N_NODES)
    dst = jnp.concatenate([ei[1], pad_dst]).reshape(NW, NCHUNK, CHUNK)

    agg1, deg = _make_sc_agg(True)(x, src, dst)
    deg = deg.reshape(NC, DEGP)

    h = _tc_dense(True, agg1, deg, x, W1l.T, b1l.reshape(1, D), W1r.T)
    (agg2,) = _make_sc_agg(False)(h, src, dst)
    out = _tc_dense(False, agg2, deg, h, W2l.T, b2l.reshape(1, D), W2r.T)
    return out


# speedup vs baseline: 1.0387x; 1.0387x over previous
import functools

import jax
import jax.numpy as jnp
from jax import lax
from jax.experimental import pallas as pl
from jax.experimental.pallas import tpu as pltpu
from jax.experimental.pallas import tpu_sc as plsc

N_NODES = 10000
N_EDGES = 320000
D = 128

NC = 2
NS = 16
NW = NC * NS
CHUNK = 128
NCHUNK = 80
HALF = NCHUNK // 2
EPW = NCHUNK * CHUNK
E_PAD = NW * EPW
NP = 10240
RPT = NP // NS
ONES_W = CHUNK
ZD_W = RPT


def _sc_agg_body(compute_deg, x_hbm, src_hbm, dst_hbm, *refs):
    if compute_deg:
        (agg_hbm, deg_hbm, src_v, dst_v, rows0_v, rows1_v, ones_v, zd_v,
         agg_sh, deg_sh, sem0, sem1, dsem) = refs
    else:
        agg_hbm, src_v, dst_v, rows0_v, rows1_v, agg_sh, sem0, sem1 = refs
    c = lax.axis_index("c")
    s = lax.axis_index("s")
    wid = c * NS + s
    base = s * RPT

    def _zrow(i, _):
        for j in range(D // 16):
            rows0_v[i, pl.ds(j * 16, 16)] = jnp.zeros((16,), jnp.float32)
        return 0
    lax.fori_loop(0, CHUNK, _zrow, 0)
    for k in range(RPT // CHUNK):
        pltpu.sync_copy(rows0_v, agg_sh.at[pl.ds(base + k * CHUNK, CHUNK)])
    if RPT % CHUNK:
        pltpu.sync_copy(
            rows0_v.at[pl.ds(0, RPT % CHUNK)],
            agg_sh.at[pl.ds(base + (RPT // CHUNK) * CHUNK, RPT % CHUNK)])
    if compute_deg:
        for j in range(ONES_W // 16):
            ones_v[pl.ds(j * 16, 16)] = jnp.ones((16,), jnp.float32)

        def _zd(i, _):
            zd_v[pl.ds(i * 16, 16)] = jnp.zeros((16,), jnp.float32)
            return 0
        lax.fori_loop(0, ZD_W // 16, _zd, 0)
        pltpu.sync_copy(zd_v.at[pl.ds(0, RPT)], deg_sh.at[pl.ds(base, RPT)])

    plsc.subcore_barrier()

    for h in range(NCHUNK // HALF):
        pltpu.sync_copy(src_hbm.at[wid, pl.ds(h * HALF, HALF)], src_v)
        pltpu.sync_copy(dst_hbm.at[wid, pl.ds(h * HALF, HALF)], dst_v)
        pltpu.async_copy(x_hbm.at[src_v.at[0]], rows0_v, sem0)
        pltpu.async_copy(x_hbm.at[src_v.at[1]], rows1_v, sem1)

        def _pair(i, _):
            j = i * 2
            for rows_v, sem, off in ((rows0_v, sem0, 0), (rows1_v, sem1, 1)):
                pltpu.make_async_copy(
                    x_hbm.at[src_v.at[j + off]], rows_v, sem).wait()
                if compute_deg:
                    pltpu.async_copy(ones_v, deg_sh.at[dst_v.at[j + off]],
                                     dsem, add=True)
                pltpu.sync_copy(rows_v, agg_sh.at[dst_v.at[j + off]], add=True)

                @pl.when(j + off + 2 < HALF)
                def _():
                    pltpu.async_copy(x_hbm.at[src_v.at[j + off + 2]], rows_v, sem)
            return 0
        lax.fori_loop(0, HALF // 2, _pair, 0)

        if compute_deg:
            def _dd(j, _):
                pltpu.make_async_copy(ones_v, deg_sh.at[dst_v.at[j]],
                                      dsem).wait()
                return 0
            lax.fori_loop(0, HALF, _dd, 0)

    plsc.subcore_barrier()
    pltpu.sync_copy(agg_sh.at[pl.ds(base, RPT)], agg_hbm.at[c, pl.ds(base, RPT)])
    if compute_deg:
        pltpu.sync_copy(deg_sh.at[pl.ds(base, RPT)],
                        deg_hbm.at[pl.ds(c * NP + base, RPT)])


def _make_sc_agg(compute_deg):
    out_type = [jax.ShapeDtypeStruct((NC, NP, D), jnp.float32)]
    if compute_deg:
        out_type.append(jax.ShapeDtypeStruct((NC * NP,), jnp.float32))
    scratch = [
        pltpu.VMEM((HALF, CHUNK), jnp.int32),
        pltpu.VMEM((HALF, CHUNK), jnp.int32),
        pltpu.VMEM((CHUNK, D), jnp.float32),
        pltpu.VMEM((CHUNK, D), jnp.float32),
    ]
    if compute_deg:
        scratch += [
            pltpu.VMEM((ONES_W,), jnp.float32),
            pltpu.VMEM((ZD_W,), jnp.float32),
        ]
    scratch += [pltpu.VMEM_SHARED((NP, D), jnp.float32)]
    if compute_deg:
        scratch += [pltpu.VMEM_SHARED((NP,), jnp.float32)]
    scratch += [pltpu.SemaphoreType.DMA] * (3 if compute_deg else 2)
    return pl.kernel(
        functools.partial(_sc_agg_body, compute_deg),
        out_type=out_type,
        mesh=plsc.VectorSubcoreMesh(
            core_axis_name="c", subcore_axis_name="s",
            num_cores=NC, num_subcores=NS,
        ),
        scratch_types=scratch,
    )


_TCB = 1024


def _tc_dense_body(relu, agg_ref, deg_ref, x_ref, wl_ref, bl_ref, wr_ref, o_ref):
    i = pl.program_id(0)
    d2 = deg_ref[:, pl.ds(i * _TCB, _TCB)]
    invd = 1.0 / jnp.maximum(d2[0] + d2[1], 1.0)
    mean = (agg_ref[0] + agg_ref[1]) * invd[:, None]
    acc = jnp.dot(mean, wl_ref[...], preferred_element_type=jnp.float32)
    acc += jnp.dot(x_ref[...], wr_ref[...], preferred_element_type=jnp.float32)
    acc += bl_ref[...]
    if relu:
        acc = jnp.maximum(acc, 0.0)
    o_ref[...] = acc


def _tc_dense(relu, agg, deg, x, wlT, bl, wrT):
    grid = -(-N_NODES // _TCB)
    return pl.pallas_call(
        functools.partial(_tc_dense_body, relu),
        grid=(grid,),
        in_specs=[
            pl.BlockSpec((NC, _TCB, D), lambda i: (0, i, 0)),
            pl.BlockSpec((NC, NP), lambda i: (0, 0)),
            pl.BlockSpec((_TCB, D), lambda i: (i, 0)),
            pl.BlockSpec((D, D), lambda i: (0, 0)),
            pl.BlockSpec((1, D), lambda i: (0, 0)),
            pl.BlockSpec((D, D), lambda i: (0, 0)),
        ],
        out_specs=pl.BlockSpec((_TCB, D), lambda i: (i, 0)),
        out_shape=jax.ShapeDtypeStruct((N_NODES, D), jnp.float32),
    )(agg, deg, x, wlT, bl, wrT)


def kernel(x, edge_index, W1l, b1l, W1r, W2l, b2l, W2r):
    ei = edge_index.astype(jnp.int32)
    npad = E_PAD - N_EDGES
    pad_src = jnp.arange(npad, dtype=jnp.int32) % N_NODES
    src = jnp.concatenate([ei[0], pad_src]).reshape(NW, NCHUNK, CHUNK)
    pad_dst = N_NODES + jnp.arange(npad, dtype=jnp.int32) % (NP - N_NODES)
    dst = jnp.concatenate([ei[1], pad_dst]).reshape(NW, NCHUNK, CHUNK)

    agg1, deg = _make_sc_agg(True)(x, src, dst)
    deg = deg.reshape(NC, NP)

    h = _tc_dense(True, agg1, deg, x, W1l.T, b1l.reshape(1, D), W1r.T)
    (agg2,) = _make_sc_agg(False)(h, src, dst)
    out = _tc_dense(False, agg2, deg, h, W2l.T, b2l.reshape(1, D), W2r.T)
    return out


# speedup vs baseline: 1.0562x; 1.0169x over previous
import functools

import jax
import jax.numpy as jnp
from jax import lax
from jax.experimental import pallas as pl
from jax.experimental.pallas import tpu as pltpu
from jax.experimental.pallas import tpu_sc as plsc

N_NODES = 10000
N_EDGES = 320000
D = 128

NC = 2
NS = 16
NW = NC * NS
CHUNK = 128
NCHUNK = 80
HALF = NCHUNK // 2
EPW = NCHUNK * CHUNK
E_PAD = NW * EPW
NP = 10240
RPT = NP // NS
ONES_W = CHUNK
ZD_W = RPT


def _sc_agg_body(compute_deg, x_hbm, src_hbm, dst_hbm, *refs):
    if compute_deg:
        (agg_hbm, deg_hbm, src_v, dst_v, rows0_v, rows1_v, ones_v, zd_v,
         agg_sh, deg_sh, sem0, sem1, dsem) = refs
    else:
        agg_hbm, src_v, dst_v, rows0_v, rows1_v, agg_sh, sem0, sem1 = refs
    c = lax.axis_index("c")
    s = lax.axis_index("s")
    wid = c * NS + s
    base = s * RPT

    def _zrow(i, _):
        for j in range(D // 16):
            rows0_v[i, pl.ds(j * 16, 16)] = jnp.zeros((16,), jnp.float32)
        return 0
    lax.fori_loop(0, CHUNK, _zrow, 0)
    for k in range(RPT // CHUNK):
        pltpu.sync_copy(rows0_v, agg_sh.at[pl.ds(base + k * CHUNK, CHUNK)])
    if RPT % CHUNK:
        pltpu.sync_copy(
            rows0_v.at[pl.ds(0, RPT % CHUNK)],
            agg_sh.at[pl.ds(base + (RPT // CHUNK) * CHUNK, RPT % CHUNK)])
    if compute_deg:
        for j in range(ONES_W // 16):
            ones_v[pl.ds(j * 16, 16)] = jnp.ones((16,), jnp.float32)

        def _zd(i, _):
            zd_v[pl.ds(i * 16, 16)] = jnp.zeros((16,), jnp.float32)
            return 0
        lax.fori_loop(0, ZD_W // 16, _zd, 0)
        pltpu.sync_copy(zd_v.at[pl.ds(0, RPT)], deg_sh.at[pl.ds(base, RPT)])

    plsc.subcore_barrier()

    for h in range(NCHUNK // HALF):
        pltpu.sync_copy(src_hbm.at[wid, pl.ds(h * HALF, HALF)], src_v)
        pltpu.sync_copy(dst_hbm.at[wid, pl.ds(h * HALF, HALF)], dst_v)
        pltpu.async_copy(x_hbm.at[src_v.at[0]], rows0_v, sem0)
        pltpu.async_copy(x_hbm.at[src_v.at[1]], rows1_v, sem1)

        def _pair(i, _):
            j = i * 2
            for rows_v, sem, off in ((rows0_v, sem0, 0), (rows1_v, sem1, 1)):
                pltpu.make_async_copy(
                    x_hbm.at[src_v.at[j + off]], rows_v, sem).wait()
                if compute_deg:
                    pltpu.async_copy(ones_v, deg_sh.at[dst_v.at[j + off]],
                                     dsem, add=True)
                pltpu.sync_copy(rows_v, agg_sh.at[dst_v.at[j + off]], add=True)

                @pl.when(j + off + 2 < HALF)
                def _():
                    pltpu.async_copy(x_hbm.at[src_v.at[j + off + 2]], rows_v, sem)
            return 0
        lax.fori_loop(0, HALF // 2, _pair, 0)

        if compute_deg:
            def _dd(j, _):
                pltpu.make_async_copy(ones_v, deg_sh.at[dst_v.at[j]],
                                      dsem).wait()
                return 0
            lax.fori_loop(0, HALF, _dd, 0)

    plsc.subcore_barrier()
    pltpu.sync_copy(agg_sh.at[pl.ds(base, RPT)], agg_hbm.at[c, pl.ds(base, RPT)])
    if compute_deg:
        pltpu.sync_copy(deg_sh.at[pl.ds(base, RPT)],
                        deg_hbm.at[pl.ds(c * NP + base, RPT)])


def _make_sc_agg(compute_deg):
    out_type = [jax.ShapeDtypeStruct((NC, NP, D), jnp.float32)]
    if compute_deg:
        out_type.append(jax.ShapeDtypeStruct((NC * NP,), jnp.float32))
    scratch = [
        pltpu.VMEM((HALF, CHUNK), jnp.int32),
        pltpu.VMEM((HALF, CHUNK), jnp.int32),
        pltpu.VMEM((CHUNK, D), jnp.float32),
        pltpu.VMEM((CHUNK, D), jnp.float32),
    ]
    if compute_deg:
        scratch += [
            pltpu.VMEM((ONES_W,), jnp.float32),
            pltpu.VMEM((ZD_W,), jnp.float32),
        ]
    scratch += [pltpu.VMEM_SHARED((NP, D), jnp.float32)]
    if compute_deg:
        scratch += [pltpu.VMEM_SHARED((NP,), jnp.float32)]
    scratch += [pltpu.SemaphoreType.DMA] * (3 if compute_deg else 2)
    return pl.kernel(
        functools.partial(_sc_agg_body, compute_deg),
        out_type=out_type,
        mesh=plsc.VectorSubcoreMesh(
            core_axis_name="c", subcore_axis_name="s",
            num_cores=NC, num_subcores=NS,
        ),
        scratch_types=scratch,
    )


_TCB = 2048


def _tc_dense_body(relu, agg_ref, deg_ref, x_ref, wl_ref, bl_ref, wr_ref, o_ref):
    i = pl.program_id(0)
    d2 = deg_ref[:, pl.ds(i * _TCB, _TCB)]
    invd = 1.0 / jnp.maximum(d2[0] + d2[1], 1.0)
    mean = (agg_ref[0] + agg_ref[1]) * invd[:, None]
    acc = jnp.dot(mean, wl_ref[...], preferred_element_type=jnp.float32)
    acc += jnp.dot(x_ref[...], wr_ref[...], preferred_element_type=jnp.float32)
    acc += bl_ref[...]
    if relu:
        acc = jnp.maximum(acc, 0.0)
    o_ref[...] = acc


def _tc_dense(relu, agg, deg, x, wlT, bl, wrT):
    grid = -(-N_NODES // _TCB)
    return pl.pallas_call(
        functools.partial(_tc_dense_body, relu),
        grid=(grid,),
        in_specs=[
            pl.BlockSpec((NC, _TCB, D), lambda i: (0, i, 0)),
            pl.BlockSpec((NC, NP), lambda i: (0, 0)),
            pl.BlockSpec((_TCB, D), lambda i: (i, 0)),
            pl.BlockSpec((D, D), lambda i: (0, 0)),
            pl.BlockSpec((1, D), lambda i: (0, 0)),
            pl.BlockSpec((D, D), lambda i: (0, 0)),
        ],
        out_specs=pl.BlockSpec((_TCB, D), lambda i: (i, 0)),
        out_shape=jax.ShapeDtypeStruct((N_NODES, D), jnp.float32),
    )(agg, deg, x, wlT, bl, wrT)


def kernel(x, edge_index, W1l, b1l, W1r, W2l, b2l, W2r):
    ei = edge_index.astype(jnp.int32)
    npad = E_PAD - N_EDGES
    pad_src = jnp.arange(npad, dtype=jnp.int32) % N_NODES
    src = jnp.concatenate([ei[0], pad_src]).reshape(NW, NCHUNK, CHUNK)
    pad_dst = N_NODES + jnp.arange(npad, dtype=jnp.int32) % (NP - N_NODES)
    dst = jnp.concatenate([ei[1], pad_dst]).reshape(NW, NCHUNK, CHUNK)

    agg1, deg = _make_sc_agg(True)(x, src, dst)
    deg = deg.reshape(NC, NP)

    h = _tc_dense(True, agg1, deg, x, W1l.T, b1l.reshape(1, D), W1r.T)
    (agg2,) = _make_sc_agg(False)(h, src, dst)
    out = _tc_dense(False, agg2, deg, h, W2l.T, b2l.reshape(1, D), W2r.T)
    return out
